# Initial kernel scaffold; baseline (speedup 1.0000x reference)
import jax
import jax.numpy as jnp
from jax.experimental import pallas as pl


def kernel(in_arrs, in_slews, c1, c2, rpi, arc_idx_r, arc_idx_f, group, unateness, delay_table, slew_table, load_index, slew_index):
    raise NotImplementedError("write your pallas kernel here")


# speedup vs baseline: 31.9332x; 31.9332x over previous
import functools

import jax
import jax.numpy as jnp
import numpy as np
from jax import lax
from jax.experimental import pallas as pl
from jax.experimental.pallas import tpu as pltpu
from jax.experimental.pallas import tpu_sc as plsc

SCALE = 1e15
NUM_ARCS = 20000
NUM_GROUPS = 10000
T_EDGES = 320000
LSE_BETA = 8.0

NW = 32
CHUNK = 256
GPAD = 320
CAP_CHUNKS = 50
EMPTY = -3.0e38
LN2 = 0.6931471805599453


def _vlog(x):
    b = plsc.bitcast(x, jnp.int32)
    e = lax.shift_right_arithmetic(b, 23) - 127
    mb = (b & 0x007FFFFF) | 0x3F800000
    m = plsc.bitcast(mb, jnp.float32)
    big = m > 1.4142135
    m = jnp.where(big, m * 0.5, m)
    ef = (e + big.astype(jnp.int32)).astype(jnp.float32)
    z = (m - 1.0) / (m + 1.0)
    z2 = z * z
    p = 2.0 * z * (1.0 + z2 * (1 / 3 + z2 * (1 / 5 + z2 * (1 / 7 + z2 * (1 / 9)))))
    return p + ef * LN2


def _sc_body(arrs_hbm, slews_hbm, c1_hbm, c2_hbm, arcr_hbm, arcf_hbm,
             group_hbm, unate_hbm, axes_hbm, corner_hbm, ebounds_hbm,
             out_hbm,
             unate_v, bnd_v, arrs_v, slews_v, c1_v, c2_v, grp_v,
             arcidx_v, cornidx_v, axes_rows, corner_rows,
             ts_v, tc_v, inarr_v, jj_v, vd_v, vs_v,
             m0, m1, m2, m3, s0t, s1t, s2t, s3t, fin_v, perm_v, sem):
    m_tabs = (m0, m1, m2, m3)
    s_tabs = (s0t, s1t, s2t, s3t)
    wid = lax.axis_index("s") * 2 + lax.axis_index("c")
    iota = lax.iota(jnp.int32, 16)
    beta = jnp.float32(LSE_BETA)

    pltpu.sync_copy(ebounds_hbm, bnd_v)
    pltpu.sync_copy(unate_hbm, unate_v)

    def _bound(w):
        return bnd_v[pl.ds(w, 16)][0]

    e0 = _bound(wid)
    e1 = _bound(wid + 1)
    gs_w = (wid * NUM_GROUPS) // NW
    a0 = (e0 // CHUNK) * CHUNK
    nch = jnp.minimum((e1 - a0 + CHUNK - 1) // CHUNK, CAP_CHUNKS)

    def init_tab(t, _):
        for c in range(4):
            m_tabs[c][pl.ds(t * 16, 16)] = jnp.full((16,), EMPTY, jnp.float32)
            s_tabs[c][pl.ds(t * 16, 16)] = jnp.zeros((16,), jnp.float32)
        return 0

    lax.fori_loop(0, GPAD // 16, init_tab, 0)

    def _edge_meta(b, u):
        eoff = u * 16
        ids = b + eoff + iota
        valid = (ids >= e0) & (ids < e1)
        g = grp_v[pl.ds(eoff, 16)]
        gl = jnp.clip(g - gs_w, 0, GPAD - 1)
        gnext = plsc.load_gather(grp_v, [eoff + jnp.minimum(iota + 1, 15)])
        islast = (g != gnext) | (iota == 15)
        return g, gl, valid, valid & islast

    def _segscan(val, g, eoff, is_sum):
        for d in (1, 2, 4, 8):
            src = jnp.maximum(iota - d, 0)
            gsrc = plsc.load_gather(grp_v, [eoff + src])
            perm_v[...] = val
            shifted = plsc.load_gather(perm_v, [src])
            merge = (iota >= d) & (gsrc == g)
            if is_sum:
                val = val + jnp.where(merge, shifted, 0.0)
            else:
                val = jnp.where(merge, jnp.maximum(val, shifted), val)
        return val

    def chunk1(k, _):
        b = a0 + k * CHUNK
        pltpu.sync_copy(arrs_hbm.at[pl.ds(b, CHUNK)], arrs_v)
        pltpu.sync_copy(slews_hbm.at[pl.ds(b, CHUNK)], slews_v)
        pltpu.sync_copy(c1_hbm.at[pl.ds(b, CHUNK)], c1_v)
        pltpu.sync_copy(c2_hbm.at[pl.ds(b, CHUNK)], c2_v)
        pltpu.sync_copy(arcr_hbm.at[pl.ds(b, CHUNK)], arcidx_v.at[pl.ds(0, CHUNK)])
        pltpu.sync_copy(arcf_hbm.at[pl.ds(b, CHUNK)], arcidx_v.at[pl.ds(CHUNK, CHUNK)])
        pltpu.sync_copy(group_hbm.at[pl.ds(b, CHUNK)], grp_v)

        cps = [pltpu.async_copy(axes_hbm.at[arcidx_v.at[pl.ds(q * 128, 128)]],
                                axes_rows.at[pl.ds(q * 128, 128)], sem)
               for q in range(2 * CHUNK // 128)]
        for cp in cps:
            cp.wait()

        def stage_c(v, _):
            p = v // 16
            r0 = v * 16
            ev0 = (v % 16) * 16
            rvec = r0 + iota
            arc = arcidx_v[pl.ds(r0, 16)]
            rf = plsc.load_gather(unate_v, [arc]) ^ p
            evec = ev0 + iota
            islew = plsc.load_gather(slews_v, [evec, rf])
            iarr = plsc.load_gather(arrs_v, [evec, rf])
            ld = (c1_v[pl.ds(ev0, 16)] + c2_v[pl.ds(ev0, 16)]) * jnp.float32(1.0 / SCALE)
            sc = jnp.zeros((16,), jnp.int32)
            for kk in range(8):
                col = jnp.full((16,), kk, jnp.int32)
                sc = sc + (plsc.load_gather(axes_rows, [rvec, col]) <= islew).astype(jnp.int32)
            ii = jnp.clip(sc - 1, 0, 6)
            cc = jnp.zeros((16,), jnp.int32)
            for kk in range(8, 16):
                col = jnp.full((16,), kk, jnp.int32)
                cc = cc + (plsc.load_gather(axes_rows, [rvec, col]) <= ld).astype(jnp.int32)
            jj = jnp.clip(cc - 1, 0, 6)
            sl0 = plsc.load_gather(axes_rows, [rvec, ii])
            sl1 = plsc.load_gather(axes_rows, [rvec, ii + 1])
            cl0 = plsc.load_gather(axes_rows, [rvec, jj + 8])
            cl1 = plsc.load_gather(axes_rows, [rvec, jj + 9])
            ts_v[pl.ds(r0, 16)] = (islew - sl0) / (sl1 - sl0)
            tc_v[pl.ds(r0, 16)] = (ld - cl0) / (cl1 - cl0)
            inarr_v[pl.ds(r0, 16)] = iarr
            jj_v[pl.ds(r0, 16)] = jj
            cornidx_v[pl.ds(r0, 16)] = arc * 7 + ii
            return 0

        lax.fori_loop(0, 2 * CHUNK // 16, stage_c, 0)

        cps = [pltpu.async_copy(corner_hbm.at[cornidx_v.at[pl.ds(q * 128, 128)]],
                                corner_rows.at[pl.ds(q * 128, 128)], sem)
               for q in range(2 * CHUNK // 128)]
        for cp in cps:
            cp.wait()

        def stage_e(v, _):
            r0 = v * 16
            rvec = r0 + iota
            ts = ts_v[pl.ds(r0, 16)]
            tc = tc_v[pl.ds(r0, 16)]
            jj = jj_v[pl.ds(r0, 16)]
            d00 = plsc.load_gather(corner_rows, [rvec, jj])
            d01 = plsc.load_gather(corner_rows, [rvec, jj + 1])
            d10 = plsc.load_gather(corner_rows, [rvec, jj + 8])
            d11 = plsc.load_gather(corner_rows, [rvec, jj + 9])
            f00 = plsc.load_gather(corner_rows, [rvec, jj + 16])
            f01 = plsc.load_gather(corner_rows, [rvec, jj + 17])
            f10 = plsc.load_gather(corner_rows, [rvec, jj + 24])
            f11 = plsc.load_gather(corner_rows, [rvec, jj + 25])
            w00 = (1.0 - ts) * (1.0 - tc)
            w01 = (1.0 - ts) * tc
            w10 = ts * (1.0 - tc)
            w11 = ts * tc
            off = k * (2 * CHUNK) + r0
            vd_v[pl.ds(off, 16)] = (w00 * d00 + w01 * d01 + w10 * d10 + w11 * d11
                                    + inarr_v[pl.ds(r0, 16)])
            vs_v[pl.ds(off, 16)] = w00 * f00 + w01 * f01 + w10 * f10 + w11 * f11
            return 0

        lax.fori_loop(0, 2 * CHUNK // 16, stage_e, 0)

        def pass1(u, _):
            eoff = u * 16
            g, gl, valid, wmask = _edge_meta(b, u)
            cb = k * (2 * CHUNK)
            cols = (vd_v[pl.ds(cb + eoff, 16)], vd_v[pl.ds(cb + CHUNK + eoff, 16)],
                    vs_v[pl.ds(cb + eoff, 16)], vs_v[pl.ds(cb + CHUNK + eoff, 16)])
            for c in range(4):
                val = _segscan(cols[c], g, eoff, is_sum=False)
                cur = plsc.load_gather(m_tabs[c], [gl])
                plsc.store_scatter(m_tabs[c], [gl], jnp.maximum(cur, val), mask=wmask)
            return 0

        lax.fori_loop(0, CHUNK // 16, pass1, 0)
        return 0

    lax.fori_loop(0, nch, chunk1, 0)

    def chunk2(k, _):
        b = a0 + k * CHUNK
        pltpu.sync_copy(group_hbm.at[pl.ds(b, CHUNK)], grp_v)

        def pass2(u, _):
            eoff = u * 16
            g, gl, valid, wmask = _edge_meta(b, u)
            cb = k * (2 * CHUNK)
            cols = (vd_v[pl.ds(cb + eoff, 16)], vd_v[pl.ds(cb + CHUNK + eoff, 16)],
                    vs_v[pl.ds(cb + eoff, 16)], vs_v[pl.ds(cb + CHUNK + eoff, 16)])
            for c in range(4):
                mg = plsc.load_gather(m_tabs[c], [gl])
                t = jnp.exp((cols[c] - mg) * beta)
                t = _segscan(jnp.where(valid, t, 0.0), g, eoff, is_sum=True)
                cur = plsc.load_gather(s_tabs[c], [gl])
                plsc.store_scatter(s_tabs[c], [gl], cur + t, mask=wmask)
            return 0

        lax.fori_loop(0, CHUNK // 16, pass2, 0)
        return 0

    lax.fori_loop(0, nch, chunk2, 0)

    def fin(t, _):
        rvec = t * 16 + iota
        for c in range(4):
            m = m_tabs[c][pl.ds(t * 16, 16)]
            s = s_tabs[c][pl.ds(t * 16, 16)]
            sh = jnp.where(m == EMPTY, 0.0, m)
            val = sh + _vlog(jnp.maximum(s, 1e-30)) * jnp.float32(1.0 / LSE_BETA)
            plsc.store_scatter(fin_v, [rvec, jnp.full((16,), c, jnp.int32)], val)
        return 0

    lax.fori_loop(0, GPAD // 16, fin, 0)
    pltpu.sync_copy(fin_v, out_hbm.at[wid])


def _make_sc_kernel():
    mesh = plsc.VectorSubcoreMesh(core_axis_name="c", subcore_axis_name="s",
                                  num_cores=2, num_subcores=16)
    scratch = [
        pltpu.VMEM((NUM_ARCS,), jnp.int32),
        pltpu.VMEM((64,), jnp.int32),
        pltpu.VMEM((CHUNK, 2), jnp.float32),
        pltpu.VMEM((CHUNK, 2), jnp.float32),
        pltpu.VMEM((CHUNK,), jnp.float32),
        pltpu.VMEM((CHUNK,), jnp.float32),
        pltpu.VMEM((CHUNK,), jnp.int32),
        pltpu.VMEM((2 * CHUNK,), jnp.int32),
        pltpu.VMEM((2 * CHUNK,), jnp.int32),
        pltpu.VMEM((2 * CHUNK, 16), jnp.float32),
        pltpu.VMEM((2 * CHUNK, 32), jnp.float32),
        pltpu.VMEM((2 * CHUNK,), jnp.float32),
        pltpu.VMEM((2 * CHUNK,), jnp.float32),
        pltpu.VMEM((2 * CHUNK,), jnp.float32),
        pltpu.VMEM((2 * CHUNK,), jnp.int32),
        pltpu.VMEM((CAP_CHUNKS * 2 * CHUNK,), jnp.float32),
        pltpu.VMEM((CAP_CHUNKS * 2 * CHUNK,), jnp.float32),
        pltpu.VMEM((GPAD,), jnp.float32),
        pltpu.VMEM((GPAD,), jnp.float32),
        pltpu.VMEM((GPAD,), jnp.float32),
        pltpu.VMEM((GPAD,), jnp.float32),
        pltpu.VMEM((GPAD,), jnp.float32),
        pltpu.VMEM((GPAD,), jnp.float32),
        pltpu.VMEM((GPAD,), jnp.float32),
        pltpu.VMEM((GPAD,), jnp.float32),
        pltpu.VMEM((GPAD, 4), jnp.float32),
        pltpu.VMEM((16,), jnp.float32),
        pltpu.SemaphoreType.DMA,
    ]
    return functools.partial(
        pl.kernel, mesh=mesh,
        out_type=jax.ShapeDtypeStruct((NW, GPAD, 4), jnp.float32),
        scratch_types=scratch,
        compiler_params=pltpu.CompilerParams(
            needs_layout_passes=False, use_tc_tiling_on_sc=False),
    )(_sc_body)


_UNPAD_ROWS = None


def _unpad_rows():
    global _UNPAD_ROWS
    if _UNPAD_ROWS is None:
        gs = (np.arange(NW + 1, dtype=np.int64) * NUM_GROUPS) // NW
        garr = np.arange(NUM_GROUPS, dtype=np.int64)
        w_of_g = np.searchsorted(gs, garr, side="right") - 1
        _UNPAD_ROWS = jnp.asarray(w_of_g * GPAD + (garr - gs[w_of_g]), dtype=jnp.int32)
    return _UNPAD_ROWS


def kernel(in_arrs, in_slews, c1, c2, rpi, arc_idx_r, arc_idx_f, group,
           unateness, delay_table, slew_table, load_index, slew_index):
    del rpi
    axes_tab = jnp.concatenate([slew_index, load_index], axis=1)
    corner_tab = jnp.concatenate(
        [delay_table[:, :7, :], delay_table[:, 1:, :],
         slew_table[:, :7, :], slew_table[:, 1:, :]], axis=2,
    ).reshape(NUM_ARCS * 7, 32)
    gs = (jnp.arange(NW + 1) * NUM_GROUPS) // NW
    e_bounds = jnp.searchsorted(group, gs.astype(group.dtype), side="left")
    e_bounds = jnp.concatenate(
        [e_bounds.astype(jnp.int32), jnp.full((31,), T_EDGES, jnp.int32)])
    padded = _make_sc_kernel()(
        in_arrs, in_slews, c1, c2,
        arc_idx_r.astype(jnp.int32), arc_idx_f.astype(jnp.int32),
        group.astype(jnp.int32), unateness.astype(jnp.int32),
        axes_tab, corner_tab, e_bounds)
    return jnp.take(padded.reshape(NW * GPAD, 4), _unpad_rows(), axis=0)


# speedup vs baseline: 49.5015x; 1.5502x over previous
import functools

import jax
import jax.numpy as jnp
import numpy as np
from jax import lax
from jax.experimental import pallas as pl
from jax.experimental.pallas import tpu as pltpu
from jax.experimental.pallas import tpu_sc as plsc

SCALE = 1e15
NUM_ARCS = 20000
NUM_GROUPS = 10000
T_EDGES = 320000
LSE_BETA = 8.0

NW = 32
CHUNK = 512
GPAD = 320
CAP_CHUNKS = 25
EMPTY = -3.0e38
LN2 = 0.6931471805599453


def _vlog(x):
    b = plsc.bitcast(x, jnp.int32)
    e = lax.shift_right_arithmetic(b, 23) - 127
    mb = (b & 0x007FFFFF) | 0x3F800000
    m = plsc.bitcast(mb, jnp.float32)
    big = m > 1.4142135
    m = jnp.where(big, m * 0.5, m)
    ef = (e + big.astype(jnp.int32)).astype(jnp.float32)
    z = (m - 1.0) / (m + 1.0)
    z2 = z * z
    p = 2.0 * z * (1.0 + z2 * (1 / 3 + z2 * (1 / 5 + z2 * (1 / 7 + z2 * (1 / 9)))))
    return p + ef * LN2


def _sc_body(pk_hbm, arcr_hbm, arcf_hbm, group_hbm, axes_hbm, corner_hbm,
             ebounds_hbm, out_hbm,
             bnd_v, pk_v, grp_v, arcidx_v, cornidx_v, axes_rows, corner_rows,
             ts_v, tc_v, inarr_v, jj_v, vd_v, vs_v,
             m0, m1, m2, m3, s0t, s1t, s2t, s3t, fin_v, perm_v, sem):
    m_tabs = (m0, m1, m2, m3)
    s_tabs = (s0t, s1t, s2t, s3t)
    wid = lax.axis_index("s") * 2 + lax.axis_index("c")
    iota = lax.iota(jnp.int32, 16)
    beta = jnp.float32(LSE_BETA)
    half = CHUNK // 16
    nvec = 2 * CHUNK // 16
    nsub = 2 * CHUNK // 128

    pltpu.sync_copy(ebounds_hbm, bnd_v)

    def _bound(w):
        return bnd_v[pl.ds(w, 16)][0]

    e0 = _bound(wid)
    e1 = _bound(wid + 1)
    gs_w = (wid * NUM_GROUPS) // NW
    a0 = (e0 // CHUNK) * CHUNK
    nch = jnp.minimum((e1 - a0 + CHUNK - 1) // CHUNK, CAP_CHUNKS)

    def init_tab(t, _):
        for c in range(4):
            m_tabs[c][pl.ds(t * 16, 16)] = jnp.full((16,), EMPTY, jnp.float32)
            s_tabs[c][pl.ds(t * 16, 16)] = jnp.zeros((16,), jnp.float32)
        return 0

    lax.fori_loop(0, GPAD // 16, init_tab, 0)

    def _edge_meta(b, u):
        eoff = u * 16
        ids = b + eoff + iota
        valid = (ids >= e0) & (ids < e1)
        g = grp_v[pl.ds(eoff, 16)]
        gl = jnp.clip(g - gs_w, 0, GPAD - 1)
        gnext = plsc.load_gather(grp_v, [eoff + jnp.minimum(iota + 1, 15)])
        islast = (g != gnext) | (iota == 15)
        return g, gl, valid, valid & islast

    def _segscan(val, g, eoff, is_sum):
        for d in (1, 2, 4, 8):
            src = jnp.maximum(iota - d, 0)
            gsrc = plsc.load_gather(grp_v, [eoff + src])
            perm_v[...] = val
            shifted = plsc.load_gather(perm_v, [src])
            merge = (iota >= d) & (gsrc == g)
            if is_sum:
                val = val + jnp.where(merge, shifted, 0.0)
            else:
                val = jnp.where(merge, jnp.maximum(val, shifted), val)
        return val

    def chunk1(k, _):
        b = a0 + k * CHUNK
        pltpu.sync_copy(pk_hbm.at[pl.ds(b, CHUNK)], pk_v)
        pltpu.sync_copy(arcr_hbm.at[pl.ds(b, CHUNK)], arcidx_v.at[pl.ds(0, CHUNK)])
        pltpu.sync_copy(arcf_hbm.at[pl.ds(b, CHUNK)], arcidx_v.at[pl.ds(CHUNK, CHUNK)])
        pltpu.sync_copy(group_hbm.at[pl.ds(b, CHUNK)], grp_v)

        cps = [pltpu.async_copy(axes_hbm.at[arcidx_v.at[pl.ds(q * 128, 128)]],
                                axes_rows.at[pl.ds(q * 128, 128)], sem)
               for q in range(nsub)]
        for cp in cps:
            cp.wait()

        def stage_c(v, _):
            p = v // half
            r0 = v * 16
            ev0 = (v % half) * 16
            rvec = r0 + iota
            arc = arcidx_v[pl.ds(r0, 16)]
            evec = ev0 + iota
            raw0 = plsc.load_gather(axes_rows, [rvec, jnp.zeros((16,), jnp.int32)])
            rf = (lax.shift_right_logical(plsc.bitcast(raw0, jnp.int32), 31)) ^ p
            islew = plsc.load_gather(pk_v, [evec, rf + 2])
            iarr = plsc.load_gather(pk_v, [evec, rf])
            ld = (plsc.load_gather(pk_v, [evec, jnp.full((16,), 4, jnp.int32)])
                  + plsc.load_gather(pk_v, [evec, jnp.full((16,), 5, jnp.int32)])
                  ) * jnp.float32(1.0 / SCALE)
            sc = (jnp.abs(raw0) <= islew).astype(jnp.int32)
            for kk in range(1, 8):
                col = jnp.full((16,), kk, jnp.int32)
                sc = sc + (plsc.load_gather(axes_rows, [rvec, col]) <= islew).astype(jnp.int32)
            ii = jnp.clip(sc - 1, 0, 6)
            cc = jnp.zeros((16,), jnp.int32)
            for kk in range(8, 16):
                col = jnp.full((16,), kk, jnp.int32)
                cc = cc + (plsc.load_gather(axes_rows, [rvec, col]) <= ld).astype(jnp.int32)
            jj = jnp.clip(cc - 1, 0, 6)
            sl0 = jnp.abs(plsc.load_gather(axes_rows, [rvec, ii]))
            sl1 = plsc.load_gather(axes_rows, [rvec, ii + 1])
            cl0 = plsc.load_gather(axes_rows, [rvec, jj + 8])
            cl1 = plsc.load_gather(axes_rows, [rvec, jj + 9])
            ts_v[pl.ds(r0, 16)] = (islew - sl0) / (sl1 - sl0)
            tc_v[pl.ds(r0, 16)] = (ld - cl0) / (cl1 - cl0)
            inarr_v[pl.ds(r0, 16)] = iarr
            jj_v[pl.ds(r0, 16)] = jj
            cornidx_v[pl.ds(r0, 16)] = arc * 7 + ii
            return 0

        lax.fori_loop(0, nvec, stage_c, 0)

        cps = [pltpu.async_copy(corner_hbm.at[cornidx_v.at[pl.ds(q * 128, 128)]],
                                corner_rows.at[pl.ds(q * 128, 128)], sem)
               for q in range(nsub)]
        for cp in cps:
            cp.wait()

        def stage_e(v, _):
            r0 = v * 16
            rvec = r0 + iota
            ts = ts_v[pl.ds(r0, 16)]
            tc = tc_v[pl.ds(r0, 16)]
            jj = jj_v[pl.ds(r0, 16)]
            d00 = plsc.load_gather(corner_rows, [rvec, jj])
            d01 = plsc.load_gather(corner_rows, [rvec, jj + 1])
            d10 = plsc.load_gather(corner_rows, [rvec, jj + 8])
            d11 = plsc.load_gather(corner_rows, [rvec, jj + 9])
            f00 = plsc.load_gather(corner_rows, [rvec, jj + 16])
            f01 = plsc.load_gather(corner_rows, [rvec, jj + 17])
            f10 = plsc.load_gather(corner_rows, [rvec, jj + 24])
            f11 = plsc.load_gather(corner_rows, [rvec, jj + 25])
            w00 = (1.0 - ts) * (1.0 - tc)
            w01 = (1.0 - ts) * tc
            w10 = ts * (1.0 - tc)
            w11 = ts * tc
            off = k * (2 * CHUNK) + r0
            vd_v[pl.ds(off, 16)] = (w00 * d00 + w01 * d01 + w10 * d10 + w11 * d11
                                    + inarr_v[pl.ds(r0, 16)])
            vs_v[pl.ds(off, 16)] = w00 * f00 + w01 * f01 + w10 * f10 + w11 * f11
            return 0

        lax.fori_loop(0, nvec, stage_e, 0)

        def pass1(u, _):
            eoff = u * 16
            g, gl, valid, wmask = _edge_meta(b, u)
            cb = k * (2 * CHUNK)
            cols = (vd_v[pl.ds(cb + eoff, 16)], vd_v[pl.ds(cb + CHUNK + eoff, 16)],
                    vs_v[pl.ds(cb + eoff, 16)], vs_v[pl.ds(cb + CHUNK + eoff, 16)])
            for c in range(4):
                val = _segscan(cols[c], g, eoff, is_sum=False)
                cur = plsc.load_gather(m_tabs[c], [gl])
                plsc.store_scatter(m_tabs[c], [gl], jnp.maximum(cur, val), mask=wmask)
            return 0

        lax.fori_loop(0, half, pass1, 0)
        return 0

    lax.fori_loop(0, nch, chunk1, 0)

    def chunk2(k, _):
        b = a0 + k * CHUNK
        pltpu.sync_copy(group_hbm.at[pl.ds(b, CHUNK)], grp_v)

        def pass2(u, _):
            eoff = u * 16
            g, gl, valid, wmask = _edge_meta(b, u)
            cb = k * (2 * CHUNK)
            cols = (vd_v[pl.ds(cb + eoff, 16)], vd_v[pl.ds(cb + CHUNK + eoff, 16)],
                    vs_v[pl.ds(cb + eoff, 16)], vs_v[pl.ds(cb + CHUNK + eoff, 16)])
            for c in range(4):
                mg = plsc.load_gather(m_tabs[c], [gl])
                t = jnp.exp((cols[c] - mg) * beta)
                t = _segscan(jnp.where(valid, t, 0.0), g, eoff, is_sum=True)
                cur = plsc.load_gather(s_tabs[c], [gl])
                plsc.store_scatter(s_tabs[c], [gl], cur + t, mask=wmask)
            return 0

        lax.fori_loop(0, half, pass2, 0)
        return 0

    lax.fori_loop(0, nch, chunk2, 0)

    def fin(t, _):
        rvec = t * 16 + iota
        for c in range(4):
            m = m_tabs[c][pl.ds(t * 16, 16)]
            s = s_tabs[c][pl.ds(t * 16, 16)]
            sh = jnp.where(m == EMPTY, 0.0, m)
            val = sh + _vlog(jnp.maximum(s, 1e-30)) * jnp.float32(1.0 / LSE_BETA)
            plsc.store_scatter(fin_v, [rvec, jnp.full((16,), c, jnp.int32)], val)
        return 0

    lax.fori_loop(0, GPAD // 16, fin, 0)
    pltpu.sync_copy(fin_v, out_hbm.at[wid])


def _make_sc_kernel():
    mesh = plsc.VectorSubcoreMesh(core_axis_name="c", subcore_axis_name="s",
                                  num_cores=2, num_subcores=16)
    scratch = [
        pltpu.VMEM((64,), jnp.int32),
        pltpu.VMEM((CHUNK, 6), jnp.float32),
        pltpu.VMEM((CHUNK,), jnp.int32),
        pltpu.VMEM((2 * CHUNK,), jnp.int32),
        pltpu.VMEM((2 * CHUNK,), jnp.int32),
        pltpu.VMEM((2 * CHUNK, 16), jnp.float32),
        pltpu.VMEM((2 * CHUNK, 32), jnp.float32),
        pltpu.VMEM((2 * CHUNK,), jnp.float32),
        pltpu.VMEM((2 * CHUNK,), jnp.float32),
        pltpu.VMEM((2 * CHUNK,), jnp.float32),
        pltpu.VMEM((2 * CHUNK,), jnp.int32),
        pltpu.VMEM((CAP_CHUNKS * 2 * CHUNK,), jnp.float32),
        pltpu.VMEM((CAP_CHUNKS * 2 * CHUNK,), jnp.float32),
        pltpu.VMEM((GPAD,), jnp.float32),
        pltpu.VMEM((GPAD,), jnp.float32),
        pltpu.VMEM((GPAD,), jnp.float32),
        pltpu.VMEM((GPAD,), jnp.float32),
        pltpu.VMEM((GPAD,), jnp.float32),
        pltpu.VMEM((GPAD,), jnp.float32),
        pltpu.VMEM((GPAD,), jnp.float32),
        pltpu.VMEM((GPAD,), jnp.float32),
        pltpu.VMEM((GPAD, 4), jnp.float32),
        pltpu.VMEM((16,), jnp.float32),
        pltpu.SemaphoreType.DMA,
    ]
    return functools.partial(
        pl.kernel, mesh=mesh,
        out_type=jax.ShapeDtypeStruct((NW, GPAD, 4), jnp.float32),
        scratch_types=scratch,
        compiler_params=pltpu.CompilerParams(
            needs_layout_passes=False, use_tc_tiling_on_sc=False),
    )(_sc_body)


_UNPAD_ROWS = None


def _unpad_rows():
    global _UNPAD_ROWS
    if _UNPAD_ROWS is None:
        gs = (np.arange(NW + 1, dtype=np.int64) * NUM_GROUPS) // NW
        garr = np.arange(NUM_GROUPS, dtype=np.int64)
        w_of_g = np.searchsorted(gs, garr, side="right") - 1
        _UNPAD_ROWS = jnp.asarray(w_of_g * GPAD + (garr - gs[w_of_g]), dtype=jnp.int32)
    return _UNPAD_ROWS


def kernel(in_arrs, in_slews, c1, c2, rpi, arc_idx_r, arc_idx_f, group,
           unateness, delay_table, slew_table, load_index, slew_index):
    del rpi
    sgn = 1.0 - 2.0 * unateness.astype(jnp.float32)
    axes_tab = jnp.concatenate(
        [slew_index[:, :1] * sgn[:, None], slew_index[:, 1:], load_index], axis=1)
    corner_tab = jnp.concatenate(
        [delay_table[:, :7, :], delay_table[:, 1:, :],
         slew_table[:, :7, :], slew_table[:, 1:, :]], axis=2,
    ).reshape(NUM_ARCS * 7, 32)
    pk = jnp.concatenate(
        [in_arrs, in_slews, c1[:, None], c2[:, None]], axis=1)
    gs = (jnp.arange(NW + 1) * NUM_GROUPS) // NW
    e_bounds = jnp.searchsorted(group, gs.astype(group.dtype), side="left")
    e_bounds = jnp.concatenate(
        [e_bounds.astype(jnp.int32), jnp.full((31,), T_EDGES, jnp.int32)])
    padded = _make_sc_kernel()(
        pk, arc_idx_r.astype(jnp.int32), arc_idx_f.astype(jnp.int32),
        group.astype(jnp.int32), axes_tab, corner_tab, e_bounds)
    return jnp.take(padded.reshape(NW * GPAD, 4), _unpad_rows(), axis=0)


# speedup vs baseline: 53.5664x; 1.0821x over previous
import functools

import jax
import jax.numpy as jnp
import numpy as np
from jax import lax
from jax.experimental import pallas as pl
from jax.experimental.pallas import tpu as pltpu
from jax.experimental.pallas import tpu_sc as plsc

SCALE = 1e15
NUM_ARCS = 20000
NUM_GROUPS = 10000
T_EDGES = 320000
LSE_BETA = 8.0

NW = 32
CHUNK = 512
GPAD = 320
CAP_CHUNKS = 25
EMPTY = -3.0e38
LN2 = 0.6931471805599453


def _vlog(x):
    b = plsc.bitcast(x, jnp.int32)
    e = lax.shift_right_arithmetic(b, 23) - 127
    mb = (b & 0x007FFFFF) | 0x3F800000
    m = plsc.bitcast(mb, jnp.float32)
    big = m > 1.4142135
    m = jnp.where(big, m * 0.5, m)
    ef = (e + big.astype(jnp.int32)).astype(jnp.float32)
    z = (m - 1.0) / (m + 1.0)
    z2 = z * z
    p = 2.0 * z * (1.0 + z2 * (1 / 3 + z2 * (1 / 5 + z2 * (1 / 7 + z2 * (1 / 9)))))
    return p + ef * LN2


def _sc_body(pk_hbm, arcr_hbm, arcf_hbm, group_hbm, axes_hbm, corner_hbm,
             ebounds_hbm, out_hbm,
             bnd_v, pk_v, grp_v, arcidx_v, cornidx_v, axes_rows, corner_rows,
             ts_v, tc_v, inarr_v, jj_v, vd_v, vs_v,
             m0, m1, m2, m3, s0t, s1t, s2t, s3t, fin_v, perm_v,
             lsem0, lsem1, lsem2, lsem3, *sems):
    m_tabs = (m0, m1, m2, m3)
    s_tabs = (s0t, s1t, s2t, s3t)
    wid = lax.axis_index("s") * 2 + lax.axis_index("c")
    iota = lax.iota(jnp.int32, 16)
    beta = jnp.float32(LSE_BETA)
    half = CHUNK // 16
    nvec = 2 * CHUNK // 16
    nsub = 2 * CHUNK // 128

    pltpu.sync_copy(ebounds_hbm, bnd_v)

    def _bound(w):
        return bnd_v[pl.ds(w, 16)][0]

    e0 = _bound(wid)
    e1 = _bound(wid + 1)
    gs_w = (wid * NUM_GROUPS) // NW
    a0 = (e0 // CHUNK) * CHUNK
    nch = jnp.minimum((e1 - a0 + CHUNK - 1) // CHUNK, CAP_CHUNKS)

    def init_tab(t, _):
        for c in range(4):
            m_tabs[c][pl.ds(t * 16, 16)] = jnp.full((16,), EMPTY, jnp.float32)
            s_tabs[c][pl.ds(t * 16, 16)] = jnp.zeros((16,), jnp.float32)
        return 0

    lax.fori_loop(0, GPAD // 16, init_tab, 0)

    def _edge_meta(b, u):
        eoff = u * 16
        ids = b + eoff + iota
        valid = (ids >= e0) & (ids < e1)
        g = grp_v[pl.ds(eoff, 16)]
        gl = jnp.clip(g - gs_w, 0, GPAD - 1)
        gnext = plsc.load_gather(grp_v, [eoff + jnp.minimum(iota + 1, 15)])
        islast = (g != gnext) | (iota == 15)
        return g, gl, valid, valid & islast

    def _segscan(val, g, eoff, is_sum):
        for d in (1, 2, 4, 8):
            src = jnp.maximum(iota - d, 0)
            gsrc = plsc.load_gather(grp_v, [eoff + src])
            perm_v[...] = val
            shifted = plsc.load_gather(perm_v, [src])
            merge = (iota >= d) & (gsrc == g)
            if is_sum:
                val = val + jnp.where(merge, shifted, 0.0)
            else:
                val = jnp.where(merge, jnp.maximum(val, shifted), val)
        return val

    def chunk1(k, _):
        b = a0 + k * CHUNK
        cpr = pltpu.async_copy(arcr_hbm.at[pl.ds(b, CHUNK)],
                               arcidx_v.at[pl.ds(0, CHUNK)], lsem0)
        cpf = pltpu.async_copy(arcf_hbm.at[pl.ds(b, CHUNK)],
                               arcidx_v.at[pl.ds(CHUNK, CHUNK)], lsem1)
        cpk = pltpu.async_copy(pk_hbm.at[pl.ds(b, CHUNK)], pk_v, lsem2)
        cpg = pltpu.async_copy(group_hbm.at[pl.ds(b, CHUNK)], grp_v, lsem3)
        cpr.wait()
        cpf.wait()

        axes_cps = [pltpu.async_copy(axes_hbm.at[arcidx_v.at[pl.ds(q * 128, 128)]],
                                     axes_rows.at[pl.ds(q * 128, 128)], sems[q])
                    for q in range(nsub)]
        cpk.wait()

        def stage_c(v, _):
            p = v // half
            r0 = v * 16
            ev0 = (v % half) * 16
            rvec = r0 + iota
            arc = arcidx_v[pl.ds(r0, 16)]
            evec = ev0 + iota
            raw0 = plsc.load_gather(axes_rows, [rvec, jnp.zeros((16,), jnp.int32)])
            rf = (lax.shift_right_logical(plsc.bitcast(raw0, jnp.int32), 31)) ^ p
            islew = plsc.load_gather(pk_v, [evec, rf + 2])
            iarr = plsc.load_gather(pk_v, [evec, rf])
            ld = (plsc.load_gather(pk_v, [evec, jnp.full((16,), 4, jnp.int32)])
                  + plsc.load_gather(pk_v, [evec, jnp.full((16,), 5, jnp.int32)])
                  ) * jnp.float32(1.0 / SCALE)
            sc = (jnp.abs(raw0) <= islew).astype(jnp.int32)
            for kk in range(1, 8):
                col = jnp.full((16,), kk, jnp.int32)
                sc = sc + (plsc.load_gather(axes_rows, [rvec, col]) <= islew).astype(jnp.int32)
            ii = jnp.clip(sc - 1, 0, 6)
            cc = jnp.zeros((16,), jnp.int32)
            for kk in range(8, 16):
                col = jnp.full((16,), kk, jnp.int32)
                cc = cc + (plsc.load_gather(axes_rows, [rvec, col]) <= ld).astype(jnp.int32)
            jj = jnp.clip(cc - 1, 0, 6)
            sl0 = jnp.abs(plsc.load_gather(axes_rows, [rvec, ii]))
            sl1 = plsc.load_gather(axes_rows, [rvec, ii + 1])
            cl0 = plsc.load_gather(axes_rows, [rvec, jj + 8])
            cl1 = plsc.load_gather(axes_rows, [rvec, jj + 9])
            ts_v[pl.ds(r0, 16)] = (islew - sl0) / (sl1 - sl0)
            tc_v[pl.ds(r0, 16)] = (ld - cl0) / (cl1 - cl0)
            inarr_v[pl.ds(r0, 16)] = iarr
            jj_v[pl.ds(r0, 16)] = jj
            cornidx_v[pl.ds(r0, 16)] = arc * 7 + ii
            return 0

        corner_cps = []
        for q in range(nsub):
            axes_cps[q].wait()
            lax.fori_loop(q * 8, q * 8 + 8, stage_c, 0)
            corner_cps.append(
                pltpu.async_copy(corner_hbm.at[cornidx_v.at[pl.ds(q * 128, 128)]],
                                 corner_rows.at[pl.ds(q * 128, 128)], sems[q]))

        def stage_e(v, _):
            r0 = v * 16
            rvec = r0 + iota
            ts = ts_v[pl.ds(r0, 16)]
            tc = tc_v[pl.ds(r0, 16)]
            jj = jj_v[pl.ds(r0, 16)]
            d00 = plsc.load_gather(corner_rows, [rvec, jj])
            d01 = plsc.load_gather(corner_rows, [rvec, jj + 1])
            d10 = plsc.load_gather(corner_rows, [rvec, jj + 8])
            d11 = plsc.load_gather(corner_rows, [rvec, jj + 9])
            f00 = plsc.load_gather(corner_rows, [rvec, jj + 16])
            f01 = plsc.load_gather(corner_rows, [rvec, jj + 17])
            f10 = plsc.load_gather(corner_rows, [rvec, jj + 24])
            f11 = plsc.load_gather(corner_rows, [rvec, jj + 25])
            w00 = (1.0 - ts) * (1.0 - tc)
            w01 = (1.0 - ts) * tc
            w10 = ts * (1.0 - tc)
            w11 = ts * tc
            off = k * (2 * CHUNK) + r0
            vd_v[pl.ds(off, 16)] = (w00 * d00 + w01 * d01 + w10 * d10 + w11 * d11
                                    + inarr_v[pl.ds(r0, 16)])
            vs_v[pl.ds(off, 16)] = w00 * f00 + w01 * f01 + w10 * f10 + w11 * f11
            return 0

        for q in range(nsub):
            corner_cps[q].wait()
            lax.fori_loop(q * 8, q * 8 + 8, stage_e, 0)
        cpg.wait()

        def pass1(u, _):
            eoff = u * 16
            g, gl, valid, wmask = _edge_meta(b, u)
            cb = k * (2 * CHUNK)
            cols = (vd_v[pl.ds(cb + eoff, 16)], vd_v[pl.ds(cb + CHUNK + eoff, 16)],
                    vs_v[pl.ds(cb + eoff, 16)], vs_v[pl.ds(cb + CHUNK + eoff, 16)])
            for c in range(4):
                val = _segscan(cols[c], g, eoff, is_sum=False)
                cur = plsc.load_gather(m_tabs[c], [gl])
                plsc.store_scatter(m_tabs[c], [gl], jnp.maximum(cur, val), mask=wmask)
            return 0

        lax.fori_loop(0, half, pass1, 0)
        return 0

    lax.fori_loop(0, nch, chunk1, 0)

    def chunk2(k, _):
        b = a0 + k * CHUNK
        pltpu.sync_copy(group_hbm.at[pl.ds(b, CHUNK)], grp_v)

        def pass2(u, _):
            eoff = u * 16
            g, gl, valid, wmask = _edge_meta(b, u)
            cb = k * (2 * CHUNK)
            cols = (vd_v[pl.ds(cb + eoff, 16)], vd_v[pl.ds(cb + CHUNK + eoff, 16)],
                    vs_v[pl.ds(cb + eoff, 16)], vs_v[pl.ds(cb + CHUNK + eoff, 16)])
            for c in range(4):
                mg = plsc.load_gather(m_tabs[c], [gl])
                t = jnp.exp((cols[c] - mg) * beta)
                t = _segscan(jnp.where(valid, t, 0.0), g, eoff, is_sum=True)
                cur = plsc.load_gather(s_tabs[c], [gl])
                plsc.store_scatter(s_tabs[c], [gl], cur + t, mask=wmask)
            return 0

        lax.fori_loop(0, half, pass2, 0)
        return 0

    lax.fori_loop(0, nch, chunk2, 0)

    def fin(t, _):
        rvec = t * 16 + iota
        for c in range(4):
            m = m_tabs[c][pl.ds(t * 16, 16)]
            s = s_tabs[c][pl.ds(t * 16, 16)]
            sh = jnp.where(m == EMPTY, 0.0, m)
            val = sh + _vlog(jnp.maximum(s, 1e-30)) * jnp.float32(1.0 / LSE_BETA)
            plsc.store_scatter(fin_v, [rvec, jnp.full((16,), c, jnp.int32)], val)
        return 0

    lax.fori_loop(0, GPAD // 16, fin, 0)
    pltpu.sync_copy(fin_v, out_hbm.at[wid])


def _make_sc_kernel():
    mesh = plsc.VectorSubcoreMesh(core_axis_name="c", subcore_axis_name="s",
                                  num_cores=2, num_subcores=16)
    scratch = [
        pltpu.VMEM((64,), jnp.int32),
        pltpu.VMEM((CHUNK, 6), jnp.float32),
        pltpu.VMEM((CHUNK,), jnp.int32),
        pltpu.VMEM((2 * CHUNK,), jnp.int32),
        pltpu.VMEM((2 * CHUNK,), jnp.int32),
        pltpu.VMEM((2 * CHUNK, 16), jnp.float32),
        pltpu.VMEM((2 * CHUNK, 32), jnp.float32),
        pltpu.VMEM((2 * CHUNK,), jnp.float32),
        pltpu.VMEM((2 * CHUNK,), jnp.float32),
        pltpu.VMEM((2 * CHUNK,), jnp.float32),
        pltpu.VMEM((2 * CHUNK,), jnp.int32),
        pltpu.VMEM((CAP_CHUNKS * 2 * CHUNK,), jnp.float32),
        pltpu.VMEM((CAP_CHUNKS * 2 * CHUNK,), jnp.float32),
        pltpu.VMEM((GPAD,), jnp.float32),
        pltpu.VMEM((GPAD,), jnp.float32),
        pltpu.VMEM((GPAD,), jnp.float32),
        pltpu.VMEM((GPAD,), jnp.float32),
        pltpu.VMEM((GPAD,), jnp.float32),
        pltpu.VMEM((GPAD,), jnp.float32),
        pltpu.VMEM((GPAD,), jnp.float32),
        pltpu.VMEM((GPAD,), jnp.float32),
        pltpu.VMEM((GPAD, 4), jnp.float32),
        pltpu.VMEM((16,), jnp.float32),
    ] + [pltpu.SemaphoreType.DMA] * (4 + 2 * CHUNK // 128)
    return functools.partial(
        pl.kernel, mesh=mesh,
        out_type=jax.ShapeDtypeStruct((NW, GPAD, 4), jnp.float32),
        scratch_types=scratch,
        compiler_params=pltpu.CompilerParams(
            needs_layout_passes=False, use_tc_tiling_on_sc=False),
    )(_sc_body)


_UNPAD_ROWS = None


def _unpad_rows():
    global _UNPAD_ROWS
    if _UNPAD_ROWS is None:
        gs = (np.arange(NW + 1, dtype=np.int64) * NUM_GROUPS) // NW
        garr = np.arange(NUM_GROUPS, dtype=np.int64)
        w_of_g = np.searchsorted(gs, garr, side="right") - 1
        _UNPAD_ROWS = jnp.asarray(w_of_g * GPAD + (garr - gs[w_of_g]), dtype=jnp.int32)
    return _UNPAD_ROWS


def kernel(in_arrs, in_slews, c1, c2, rpi, arc_idx_r, arc_idx_f, group,
           unateness, delay_table, slew_table, load_index, slew_index):
    del rpi
    sgn = 1.0 - 2.0 * unateness.astype(jnp.float32)
    axes_tab = jnp.concatenate(
        [slew_index[:, :1] * sgn[:, None], slew_index[:, 1:], load_index], axis=1)
    corner_tab = jnp.concatenate(
        [delay_table[:, :7, :], delay_table[:, 1:, :],
         slew_table[:, :7, :], slew_table[:, 1:, :]], axis=2,
    ).reshape(NUM_ARCS * 7, 32)
    pk = jnp.concatenate(
        [in_arrs, in_slews, c1[:, None], c2[:, None]], axis=1)
    gs = (jnp.arange(NW + 1) * NUM_GROUPS) // NW
    e_bounds = jnp.searchsorted(group, gs.astype(group.dtype), side="left")
    e_bounds = jnp.concatenate(
        [e_bounds.astype(jnp.int32), jnp.full((31,), T_EDGES, jnp.int32)])
    padded = _make_sc_kernel()(
        pk, arc_idx_r.astype(jnp.int32), arc_idx_f.astype(jnp.int32),
        group.astype(jnp.int32), axes_tab, corner_tab, e_bounds)
    return jnp.take(padded.reshape(NW * GPAD, 4), _unpad_rows(), axis=0)


# speedup vs baseline: 60.5682x; 1.1307x over previous
import functools

import jax
import jax.numpy as jnp
import numpy as np
from jax import lax
from jax.experimental import pallas as pl
from jax.experimental.pallas import tpu as pltpu
from jax.experimental.pallas import tpu_sc as plsc

SCALE = 1e15
NUM_ARCS = 20000
NUM_GROUPS = 10000
T_EDGES = 320000
LSE_BETA = 8.0

NW = 32
CHUNK = 512
GPAD = 320
CAP_CHUNKS = 25
EMPTY = -3.0e38
LN2 = 0.6931471805599453


def _vlog(x):
    b = plsc.bitcast(x, jnp.int32)
    e = lax.shift_right_arithmetic(b, 23) - 127
    mb = (b & 0x007FFFFF) | 0x3F800000
    m = plsc.bitcast(mb, jnp.float32)
    big = m > 1.4142135
    m = jnp.where(big, m * 0.5, m)
    ef = (e + big.astype(jnp.int32)).astype(jnp.float32)
    z = (m - 1.0) / (m + 1.0)
    z2 = z * z
    p = 2.0 * z * (1.0 + z2 * (1 / 3 + z2 * (1 / 5 + z2 * (1 / 7 + z2 * (1 / 9)))))
    return p + ef * LN2


def _sc_body(pk_hbm, arcr_hbm, arcf_hbm, group_hbm, axes_hbm, corner_hbm,
             ebounds_hbm, out_hbm,
             bnd_v, pk_v, grp_v, arcidx_v, cornidx_v, axes_rows, corner_rows,
             ts_v, tc_v, inarr_v, jj_v, vd_v, vs_v,
             m0, m1, m2, m3, s0t, s1t, s2t, s3t, fin_v, perm_v,
             lsem0, lsem1, lsem2, lsem3, *sems):
    m_tabs = (m0, m1, m2, m3)
    s_tabs = (s0t, s1t, s2t, s3t)
    wid = lax.axis_index("s") * 2 + lax.axis_index("c")
    iota = lax.iota(jnp.int32, 16)
    beta = jnp.float32(LSE_BETA)
    half = CHUNK // 16
    nvec = 2 * CHUNK // 16
    nsub = 2 * CHUNK // 128

    pltpu.sync_copy(ebounds_hbm, bnd_v)

    def _bound(w):
        return bnd_v[pl.ds(w, 16)][0]

    e0 = _bound(wid)
    e1 = _bound(wid + 1)
    gs_w = (wid * NUM_GROUPS) // NW
    a0 = (e0 // CHUNK) * CHUNK
    nch = jnp.minimum((e1 - a0 + CHUNK - 1) // CHUNK, CAP_CHUNKS)

    def init_tab(t, _):
        for c in range(4):
            m_tabs[c][pl.ds(t * 16, 16)] = jnp.full((16,), EMPTY, jnp.float32)
            s_tabs[c][pl.ds(t * 16, 16)] = jnp.zeros((16,), jnp.float32)
        return 0

    lax.fori_loop(0, GPAD // 16, init_tab, 0)

    def _edge_meta(b, u):
        eoff = u * 16
        ids = b + eoff + iota
        valid = (ids >= e0) & (ids < e1)
        g = grp_v[pl.ds(eoff, 16)]
        gl = jnp.clip(g - gs_w, 0, GPAD - 1)
        gnext = plsc.load_gather(grp_v, [eoff + jnp.minimum(iota + 1, 15)])
        islast = (g != gnext) | (iota == 15)
        return g, gl, valid, valid & islast

    def _segscan4(vals, g, eoff, is_sum):
        vals = list(vals)
        for d in (1, 2, 4, 8):
            src = jnp.maximum(iota - d, 0)
            gsrc = plsc.load_gather(grp_v, [eoff + src])
            merge = (iota >= d) & (gsrc == g)
            for c in range(4):
                perm_v[pl.ds(c * 16, 16)] = vals[c]
            for c in range(4):
                shifted = plsc.load_gather(perm_v, [c * 16 + src])
                if is_sum:
                    vals[c] = vals[c] + jnp.where(merge, shifted, 0.0)
                else:
                    vals[c] = jnp.where(merge, jnp.maximum(vals[c], shifted), vals[c])
        return vals

    def chunk1(k, _):
        b = a0 + k * CHUNK
        cpr = pltpu.async_copy(arcr_hbm.at[pl.ds(b, CHUNK)],
                               arcidx_v.at[pl.ds(0, CHUNK)], lsem0)
        cpf = pltpu.async_copy(arcf_hbm.at[pl.ds(b, CHUNK)],
                               arcidx_v.at[pl.ds(CHUNK, CHUNK)], lsem1)
        cpk = pltpu.async_copy(pk_hbm.at[pl.ds(b, CHUNK)], pk_v, lsem2)
        cpg = pltpu.async_copy(group_hbm.at[pl.ds(b, CHUNK)], grp_v, lsem3)
        cpr.wait()
        cpf.wait()

        axes_cps = [pltpu.async_copy(axes_hbm.at[arcidx_v.at[pl.ds(q * 128, 128)]],
                                     axes_rows.at[pl.ds(q * 128, 128)], sems[q])
                    for q in range(nsub)]
        cpk.wait()

        def stage_c(v, _):
            p = v // half
            r0 = v * 16
            ev0 = (v % half) * 16
            rvec = r0 + iota
            arc = arcidx_v[pl.ds(r0, 16)]
            evec = ev0 + iota
            raw0 = plsc.load_gather(axes_rows, [rvec, jnp.zeros((16,), jnp.int32)])
            rf = (lax.shift_right_logical(plsc.bitcast(raw0, jnp.int32), 31)) ^ p
            islew = plsc.load_gather(pk_v, [evec, rf + 2])
            iarr = plsc.load_gather(pk_v, [evec, rf])
            ld = (plsc.load_gather(pk_v, [evec, jnp.full((16,), 4, jnp.int32)])
                  + plsc.load_gather(pk_v, [evec, jnp.full((16,), 5, jnp.int32)])
                  ) * jnp.float32(1.0 / SCALE)
            sc = (jnp.abs(raw0) <= islew).astype(jnp.int32)
            for kk in range(1, 8):
                col = jnp.full((16,), kk, jnp.int32)
                sc = sc + (plsc.load_gather(axes_rows, [rvec, col]) <= islew).astype(jnp.int32)
            ii = jnp.clip(sc - 1, 0, 6)
            cc = jnp.zeros((16,), jnp.int32)
            for kk in range(8, 16):
                col = jnp.full((16,), kk, jnp.int32)
                cc = cc + (plsc.load_gather(axes_rows, [rvec, col]) <= ld).astype(jnp.int32)
            jj = jnp.clip(cc - 1, 0, 6)
            sl0 = jnp.abs(plsc.load_gather(axes_rows, [rvec, ii]))
            sl1 = plsc.load_gather(axes_rows, [rvec, ii + 1])
            cl0 = plsc.load_gather(axes_rows, [rvec, jj + 8])
            cl1 = plsc.load_gather(axes_rows, [rvec, jj + 9])
            ts_v[pl.ds(r0, 16)] = (islew - sl0) / (sl1 - sl0)
            tc_v[pl.ds(r0, 16)] = (ld - cl0) / (cl1 - cl0)
            inarr_v[pl.ds(r0, 16)] = iarr
            jj_v[pl.ds(r0, 16)] = jj
            cornidx_v[pl.ds(r0, 16)] = arc * 7 + ii
            return 0

        corner_cps = []
        for q in range(nsub):
            axes_cps[q].wait()
            lax.fori_loop(q * 8, q * 8 + 8, stage_c, 0)
            corner_cps.append(
                pltpu.async_copy(corner_hbm.at[cornidx_v.at[pl.ds(q * 128, 128)]],
                                 corner_rows.at[pl.ds(q * 128, 128)], sems[q]))

        def stage_e(v, _):
            r0 = v * 16
            rvec = r0 + iota
            ts = ts_v[pl.ds(r0, 16)]
            tc = tc_v[pl.ds(r0, 16)]
            jj = jj_v[pl.ds(r0, 16)]
            d00 = plsc.load_gather(corner_rows, [rvec, jj])
            d01 = plsc.load_gather(corner_rows, [rvec, jj + 1])
            d10 = plsc.load_gather(corner_rows, [rvec, jj + 8])
            d11 = plsc.load_gather(corner_rows, [rvec, jj + 9])
            f00 = plsc.load_gather(corner_rows, [rvec, jj + 16])
            f01 = plsc.load_gather(corner_rows, [rvec, jj + 17])
            f10 = plsc.load_gather(corner_rows, [rvec, jj + 24])
            f11 = plsc.load_gather(corner_rows, [rvec, jj + 25])
            w00 = (1.0 - ts) * (1.0 - tc)
            w01 = (1.0 - ts) * tc
            w10 = ts * (1.0 - tc)
            w11 = ts * tc
            off = k * (2 * CHUNK) + r0
            vd_v[pl.ds(off, 16)] = (w00 * d00 + w01 * d01 + w10 * d10 + w11 * d11
                                    + inarr_v[pl.ds(r0, 16)])
            vs_v[pl.ds(off, 16)] = w00 * f00 + w01 * f01 + w10 * f10 + w11 * f11
            return 0

        for q in range(nsub):
            corner_cps[q].wait()
            lax.fori_loop(q * 8, q * 8 + 8, stage_e, 0)
        cpg.wait()

        def pass1(u, _):
            eoff = u * 16
            g, gl, valid, wmask = _edge_meta(b, u)
            cb = k * (2 * CHUNK)
            cols = (vd_v[pl.ds(cb + eoff, 16)], vd_v[pl.ds(cb + CHUNK + eoff, 16)],
                    vs_v[pl.ds(cb + eoff, 16)], vs_v[pl.ds(cb + CHUNK + eoff, 16)])
            vals = _segscan4(cols, g, eoff, is_sum=False)
            for c in range(4):
                cur = plsc.load_gather(m_tabs[c], [gl])
                plsc.store_scatter(m_tabs[c], [gl], jnp.maximum(cur, vals[c]), mask=wmask)
            return 0

        lax.fori_loop(0, half, pass1, 0)
        return 0

    lax.fori_loop(0, nch, chunk1, 0)

    def chunk2(k, _):
        b = a0 + k * CHUNK
        pltpu.sync_copy(group_hbm.at[pl.ds(b, CHUNK)], grp_v)

        def pass2(u, _):
            eoff = u * 16
            g, gl, valid, wmask = _edge_meta(b, u)
            cb = k * (2 * CHUNK)
            cols = (vd_v[pl.ds(cb + eoff, 16)], vd_v[pl.ds(cb + CHUNK + eoff, 16)],
                    vs_v[pl.ds(cb + eoff, 16)], vs_v[pl.ds(cb + CHUNK + eoff, 16)])
            ts = [jnp.where(valid,
                            jnp.exp((cols[c] - plsc.load_gather(m_tabs[c], [gl])) * beta),
                            0.0) for c in range(4)]
            vals = _segscan4(ts, g, eoff, is_sum=True)
            for c in range(4):
                cur = plsc.load_gather(s_tabs[c], [gl])
                plsc.store_scatter(s_tabs[c], [gl], cur + vals[c], mask=wmask)
            return 0

        lax.fori_loop(0, half, pass2, 0)
        return 0

    lax.fori_loop(0, nch, chunk2, 0)

    def fin(t, _):
        rvec = t * 16 + iota
        for c in range(4):
            m = m_tabs[c][pl.ds(t * 16, 16)]
            s = s_tabs[c][pl.ds(t * 16, 16)]
            sh = jnp.where(m == EMPTY, 0.0, m)
            val = sh + _vlog(jnp.maximum(s, 1e-30)) * jnp.float32(1.0 / LSE_BETA)
            plsc.store_scatter(fin_v, [rvec, jnp.full((16,), c, jnp.int32)], val)
        return 0

    lax.fori_loop(0, GPAD // 16, fin, 0)
    pltpu.sync_copy(fin_v, out_hbm.at[wid])


def _make_sc_kernel():
    mesh = plsc.VectorSubcoreMesh(core_axis_name="c", subcore_axis_name="s",
                                  num_cores=2, num_subcores=16)
    scratch = [
        pltpu.VMEM((64,), jnp.int32),
        pltpu.VMEM((CHUNK, 6), jnp.float32),
        pltpu.VMEM((CHUNK,), jnp.int32),
        pltpu.VMEM((2 * CHUNK,), jnp.int32),
        pltpu.VMEM((2 * CHUNK,), jnp.int32),
        pltpu.VMEM((2 * CHUNK, 16), jnp.float32),
        pltpu.VMEM((2 * CHUNK, 32), jnp.float32),
        pltpu.VMEM((2 * CHUNK,), jnp.float32),
        pltpu.VMEM((2 * CHUNK,), jnp.float32),
        pltpu.VMEM((2 * CHUNK,), jnp.float32),
        pltpu.VMEM((2 * CHUNK,), jnp.int32),
        pltpu.VMEM((CAP_CHUNKS * 2 * CHUNK,), jnp.float32),
        pltpu.VMEM((CAP_CHUNKS * 2 * CHUNK,), jnp.float32),
        pltpu.VMEM((GPAD,), jnp.float32),
        pltpu.VMEM((GPAD,), jnp.float32),
        pltpu.VMEM((GPAD,), jnp.float32),
        pltpu.VMEM((GPAD,), jnp.float32),
        pltpu.VMEM((GPAD,), jnp.float32),
        pltpu.VMEM((GPAD,), jnp.float32),
        pltpu.VMEM((GPAD,), jnp.float32),
        pltpu.VMEM((GPAD,), jnp.float32),
        pltpu.VMEM((GPAD, 4), jnp.float32),
        pltpu.VMEM((64,), jnp.float32),
    ] + [pltpu.SemaphoreType.DMA] * (4 + 2 * CHUNK // 128)
    return functools.partial(
        pl.kernel, mesh=mesh,
        out_type=jax.ShapeDtypeStruct((NW, GPAD, 4), jnp.float32),
        scratch_types=scratch,
        compiler_params=pltpu.CompilerParams(
            needs_layout_passes=False, use_tc_tiling_on_sc=False),
    )(_sc_body)


_UNPAD_ROWS = None


def _unpad_rows():
    global _UNPAD_ROWS
    if _UNPAD_ROWS is None:
        gs = (np.arange(NW + 1, dtype=np.int64) * NUM_GROUPS) // NW
        garr = np.arange(NUM_GROUPS, dtype=np.int64)
        w_of_g = np.searchsorted(gs, garr, side="right") - 1
        _UNPAD_ROWS = jnp.asarray(w_of_g * GPAD + (garr - gs[w_of_g]), dtype=jnp.int32)
    return _UNPAD_ROWS


def kernel(in_arrs, in_slews, c1, c2, rpi, arc_idx_r, arc_idx_f, group,
           unateness, delay_table, slew_table, load_index, slew_index):
    del rpi
    sgn = 1.0 - 2.0 * unateness.astype(jnp.float32)
    axes_tab = jnp.concatenate(
        [slew_index[:, :1] * sgn[:, None], slew_index[:, 1:], load_index], axis=1)
    corner_tab = jnp.concatenate(
        [delay_table[:, :7, :], delay_table[:, 1:, :],
         slew_table[:, :7, :], slew_table[:, 1:, :]], axis=2,
    ).reshape(NUM_ARCS * 7, 32)
    pk = jnp.concatenate(
        [in_arrs, in_slews, c1[:, None], c2[:, None]], axis=1)
    gs = (jnp.arange(NW + 1) * NUM_GROUPS) // NW
    e_bounds = jnp.searchsorted(group, gs.astype(group.dtype), side="left")
    e_bounds = jnp.concatenate(
        [e_bounds.astype(jnp.int32), jnp.full((31,), T_EDGES, jnp.int32)])
    padded = _make_sc_kernel()(
        pk, arc_idx_r.astype(jnp.int32), arc_idx_f.astype(jnp.int32),
        group.astype(jnp.int32), axes_tab, corner_tab, e_bounds)
    return jnp.take(padded.reshape(NW * GPAD, 4), _unpad_rows(), axis=0)


# speedup vs baseline: 61.6127x; 1.0172x over previous
import functools

import jax
import jax.numpy as jnp
import numpy as np
from jax import lax
from jax.experimental import pallas as pl
from jax.experimental.pallas import tpu as pltpu
from jax.experimental.pallas import tpu_sc as plsc

SCALE = 1e15
NUM_ARCS = 20000
NUM_GROUPS = 10000
T_EDGES = 320000
LSE_BETA = 8.0

NW = 32
CHUNK = 512
GPAD = 320
CAP_CHUNKS = 25
EMPTY = -3.0e38
LN2 = 0.6931471805599453


def _vlog(x):
    b = plsc.bitcast(x, jnp.int32)
    e = lax.shift_right_arithmetic(b, 23) - 127
    mb = (b & 0x007FFFFF) | 0x3F800000
    m = plsc.bitcast(mb, jnp.float32)
    big = m > 1.4142135
    m = jnp.where(big, m * 0.5, m)
    ef = (e + big.astype(jnp.int32)).astype(jnp.float32)
    z = (m - 1.0) / (m + 1.0)
    z2 = z * z
    p = 2.0 * z * (1.0 + z2 * (1 / 3 + z2 * (1 / 5 + z2 * (1 / 7 + z2 * (1 / 9)))))
    return p + ef * LN2


def _sc_body(pk_hbm, arcr_hbm, arcf_hbm, group_hbm, axes_hbm, corner_hbm,
             ebounds_hbm, out_hbm,
             bnd_v, pk_v, grp_v, arcidx_v, cornidx_v, axes_rows, corner_rows,
             ts_v, tc_v, inarr_v, jj_v, vd_v, vs_v,
             m0, m1, m2, m3, s0t, s1t, s2t, s3t, fin_v, perm_v,
             lsem0, lsem1, lsem2, lsem3, *sems):
    m_tabs = (m0, m1, m2, m3)
    s_tabs = (s0t, s1t, s2t, s3t)
    wid = lax.axis_index("s") * 2 + lax.axis_index("c")
    iota = lax.iota(jnp.int32, 16)
    beta = jnp.float32(LSE_BETA)
    nsub = 2 * CHUNK // 128
    half = CHUNK // 16

    pltpu.sync_copy(ebounds_hbm, bnd_v)

    def _bound(w):
        return bnd_v[pl.ds(w, 16)][0]

    e0 = _bound(wid)
    e1 = _bound(wid + 1)
    gs_w = (wid * NUM_GROUPS) // NW
    a0 = (e0 // CHUNK) * CHUNK
    nch = jnp.minimum((e1 - a0 + CHUNK - 1) // CHUNK, CAP_CHUNKS)

    def init_tab(t, _):
        for c in range(4):
            m_tabs[c][pl.ds(t * 16, 16)] = jnp.full((16,), EMPTY, jnp.float32)
            s_tabs[c][pl.ds(t * 16, 16)] = jnp.zeros((16,), jnp.float32)
        return 0

    lax.fori_loop(0, GPAD // 16, init_tab, 0)

    def _issue_lin(kk, par):
        bq = a0 + kk * CHUNK
        pltpu.async_copy(arcr_hbm.at[pl.ds(bq, CHUNK)],
                         arcidx_v.at[pl.ds(par * 2 * CHUNK, CHUNK)], lsem0)
        pltpu.async_copy(arcf_hbm.at[pl.ds(bq, CHUNK)],
                         arcidx_v.at[pl.ds(par * 2 * CHUNK + CHUNK, CHUNK)], lsem1)
        pltpu.async_copy(pk_hbm.at[pl.ds(bq, CHUNK)],
                         pk_v.at[pl.ds(par * CHUNK, CHUNK)], lsem2)
        pltpu.async_copy(group_hbm.at[pl.ds(bq, CHUNK)],
                         grp_v.at[pl.ds(par * CHUNK, CHUNK)], lsem3)

    def _wait_lin():
        pltpu.make_async_copy(arcr_hbm.at[pl.ds(0, CHUNK)],
                              arcidx_v.at[pl.ds(0, CHUNK)], lsem0).wait()
        pltpu.make_async_copy(arcf_hbm.at[pl.ds(0, CHUNK)],
                              arcidx_v.at[pl.ds(0, CHUNK)], lsem1).wait()
        pltpu.make_async_copy(pk_hbm.at[pl.ds(0, CHUNK)],
                              pk_v.at[pl.ds(0, CHUNK)], lsem2).wait()
        pltpu.make_async_copy(group_hbm.at[pl.ds(0, CHUNK)],
                              grp_v.at[pl.ds(0, CHUNK)], lsem3).wait()

    def _edge_meta(b, gbase, u):
        eoff = u * 16
        ids = b + eoff + iota
        valid = (ids >= e0) & (ids < e1)
        g = grp_v[pl.ds(gbase + eoff, 16)]
        gl = jnp.clip(g - gs_w, 0, GPAD - 1)
        gnext = plsc.load_gather(grp_v, [gbase + eoff + jnp.minimum(iota + 1, 15)])
        islast = (g != gnext) | (iota == 15)
        return g, gl, valid, valid & islast

    def _segscan4(vals, g, gbase, eoff, is_sum):
        vals = list(vals)
        for d in (1, 2, 4, 8):
            src = jnp.maximum(iota - d, 0)
            gsrc = plsc.load_gather(grp_v, [gbase + eoff + src])
            merge = (iota >= d) & (gsrc == g)
            for c in range(4):
                perm_v[pl.ds(c * 16, 16)] = vals[c]
            for c in range(4):
                shifted = plsc.load_gather(perm_v, [c * 16 + src])
                if is_sum:
                    vals[c] = vals[c] + jnp.where(merge, shifted, 0.0)
                else:
                    vals[c] = jnp.where(merge, jnp.maximum(vals[c], shifted), vals[c])
        return vals

    pl.when(nch > 0)(lambda: _issue_lin(0, 0))

    def chunk1(k, _):
        b = a0 + k * CHUNK
        par = lax.rem(k, 2)
        abase = par * 2 * CHUNK
        gbase = par * CHUNK
        _wait_lin()

        axes_cps = [pltpu.async_copy(
            axes_hbm.at[arcidx_v.at[pl.ds(abase + q * 128, 128)]],
            axes_rows.at[pl.ds(q * 128, 128)], sems[q])
            for q in range(nsub)]

        pl.when(k + 1 < nch)(lambda: _issue_lin(k + 1, 1 - par))

        def stage_c(v):
            p = v // half
            r0 = v * 16
            ev0 = gbase + (v % half) * 16
            rvec = r0 + iota
            arc = arcidx_v[pl.ds(abase + r0, 16)]
            evec = ev0 + iota
            raw0 = plsc.load_gather(axes_rows, [rvec, jnp.zeros((16,), jnp.int32)])
            rf = (lax.shift_right_logical(plsc.bitcast(raw0, jnp.int32), 31)) ^ p
            islew = plsc.load_gather(pk_v, [evec, rf + 2])
            iarr = plsc.load_gather(pk_v, [evec, rf])
            ld = (plsc.load_gather(pk_v, [evec, jnp.full((16,), 4, jnp.int32)])
                  + plsc.load_gather(pk_v, [evec, jnp.full((16,), 5, jnp.int32)])
                  ) * jnp.float32(1.0 / SCALE)
            sc = (jnp.abs(raw0) <= islew).astype(jnp.int32)
            for kk in range(1, 8):
                col = jnp.full((16,), kk, jnp.int32)
                sc = sc + (plsc.load_gather(axes_rows, [rvec, col]) <= islew).astype(jnp.int32)
            ii = jnp.clip(sc - 1, 0, 6)
            cc = jnp.zeros((16,), jnp.int32)
            for kk in range(8, 16):
                col = jnp.full((16,), kk, jnp.int32)
                cc = cc + (plsc.load_gather(axes_rows, [rvec, col]) <= ld).astype(jnp.int32)
            jj = jnp.clip(cc - 1, 0, 6)
            sl0 = jnp.abs(plsc.load_gather(axes_rows, [rvec, ii]))
            sl1 = plsc.load_gather(axes_rows, [rvec, ii + 1])
            cl0 = plsc.load_gather(axes_rows, [rvec, jj + 8])
            cl1 = plsc.load_gather(axes_rows, [rvec, jj + 9])
            ts_v[pl.ds(r0, 16)] = (islew - sl0) / (sl1 - sl0)
            tc_v[pl.ds(r0, 16)] = (ld - cl0) / (cl1 - cl0)
            inarr_v[pl.ds(r0, 16)] = iarr
            jj_v[pl.ds(r0, 16)] = jj
            cornidx_v[pl.ds(r0, 16)] = arc * 7 + ii

        corner_cps = []
        for q in range(nsub):
            axes_cps[q].wait()
            plsc.parallel_loop(q * 8, q * 8 + 8, unroll=2)(stage_c)
            corner_cps.append(
                pltpu.async_copy(corner_hbm.at[cornidx_v.at[pl.ds(q * 128, 128)]],
                                 corner_rows.at[pl.ds(q * 128, 128)], sems[q]))

        def stage_e(v):
            r0 = v * 16
            rvec = r0 + iota
            ts = ts_v[pl.ds(r0, 16)]
            tc = tc_v[pl.ds(r0, 16)]
            jj = jj_v[pl.ds(r0, 16)]
            d00 = plsc.load_gather(corner_rows, [rvec, jj])
            d01 = plsc.load_gather(corner_rows, [rvec, jj + 1])
            d10 = plsc.load_gather(corner_rows, [rvec, jj + 8])
            d11 = plsc.load_gather(corner_rows, [rvec, jj + 9])
            f00 = plsc.load_gather(corner_rows, [rvec, jj + 16])
            f01 = plsc.load_gather(corner_rows, [rvec, jj + 17])
            f10 = plsc.load_gather(corner_rows, [rvec, jj + 24])
            f11 = plsc.load_gather(corner_rows, [rvec, jj + 25])
            w00 = (1.0 - ts) * (1.0 - tc)
            w01 = (1.0 - ts) * tc
            w10 = ts * (1.0 - tc)
            w11 = ts * tc
            off = k * (2 * CHUNK) + r0
            vd_v[pl.ds(off, 16)] = (w00 * d00 + w01 * d01 + w10 * d10 + w11 * d11
                                    + inarr_v[pl.ds(r0, 16)])
            vs_v[pl.ds(off, 16)] = w00 * f00 + w01 * f01 + w10 * f10 + w11 * f11

        for q in range(nsub):
            corner_cps[q].wait()
            plsc.parallel_loop(q * 8, q * 8 + 8, unroll=2)(stage_e)

        def pass1(u, _):
            eoff = u * 16
            g, gl, valid, wmask = _edge_meta(b, gbase, u)
            cb = k * (2 * CHUNK)
            cols = (vd_v[pl.ds(cb + eoff, 16)], vd_v[pl.ds(cb + CHUNK + eoff, 16)],
                    vs_v[pl.ds(cb + eoff, 16)], vs_v[pl.ds(cb + CHUNK + eoff, 16)])
            vals = _segscan4(cols, g, gbase, eoff, is_sum=False)
            for c in range(4):
                cur = plsc.load_gather(m_tabs[c], [gl])
                plsc.store_scatter(m_tabs[c], [gl], jnp.maximum(cur, vals[c]), mask=wmask)
            return 0

        lax.fori_loop(0, half, pass1, 0)
        return 0

    lax.fori_loop(0, nch, chunk1, 0)

    def chunk2(k, _):
        b = a0 + k * CHUNK
        pltpu.sync_copy(group_hbm.at[pl.ds(b, CHUNK)], grp_v.at[pl.ds(0, CHUNK)])

        def pass2(u, _):
            eoff = u * 16
            g, gl, valid, wmask = _edge_meta(b, 0, u)
            cb = k * (2 * CHUNK)
            cols = (vd_v[pl.ds(cb + eoff, 16)], vd_v[pl.ds(cb + CHUNK + eoff, 16)],
                    vs_v[pl.ds(cb + eoff, 16)], vs_v[pl.ds(cb + CHUNK + eoff, 16)])
            ts = [jnp.where(valid,
                            jnp.exp((cols[c] - plsc.load_gather(m_tabs[c], [gl])) * beta),
                            0.0) for c in range(4)]
            vals = _segscan4(ts, g, 0, eoff, is_sum=True)
            for c in range(4):
                cur = plsc.load_gather(s_tabs[c], [gl])
                plsc.store_scatter(s_tabs[c], [gl], cur + vals[c], mask=wmask)
            return 0

        lax.fori_loop(0, half, pass2, 0)
        return 0

    lax.fori_loop(0, nch, chunk2, 0)

    def fin(t, _):
        rvec = t * 16 + iota
        for c in range(4):
            m = m_tabs[c][pl.ds(t * 16, 16)]
            s = s_tabs[c][pl.ds(t * 16, 16)]
            sh = jnp.where(m == EMPTY, 0.0, m)
            val = sh + _vlog(jnp.maximum(s, 1e-30)) * jnp.float32(1.0 / LSE_BETA)
            plsc.store_scatter(fin_v, [rvec, jnp.full((16,), c, jnp.int32)], val)
        return 0

    lax.fori_loop(0, GPAD // 16, fin, 0)
    pltpu.sync_copy(fin_v, out_hbm.at[wid])


def _make_sc_kernel():
    mesh = plsc.VectorSubcoreMesh(core_axis_name="c", subcore_axis_name="s",
                                  num_cores=2, num_subcores=16)
    scratch = [
        pltpu.VMEM((64,), jnp.int32),
        pltpu.VMEM((2 * CHUNK, 6), jnp.float32),
        pltpu.VMEM((2 * CHUNK,), jnp.int32),
        pltpu.VMEM((4 * CHUNK,), jnp.int32),
        pltpu.VMEM((2 * CHUNK,), jnp.int32),
        pltpu.VMEM((2 * CHUNK, 16), jnp.float32),
        pltpu.VMEM((2 * CHUNK, 32), jnp.float32),
        pltpu.VMEM((2 * CHUNK,), jnp.float32),
        pltpu.VMEM((2 * CHUNK,), jnp.float32),
        pltpu.VMEM((2 * CHUNK,), jnp.float32),
        pltpu.VMEM((2 * CHUNK,), jnp.int32),
        pltpu.VMEM((CAP_CHUNKS * 2 * CHUNK,), jnp.float32),
        pltpu.VMEM((CAP_CHUNKS * 2 * CHUNK,), jnp.float32),
        pltpu.VMEM((GPAD,), jnp.float32),
        pltpu.VMEM((GPAD,), jnp.float32),
        pltpu.VMEM((GPAD,), jnp.float32),
        pltpu.VMEM((GPAD,), jnp.float32),
        pltpu.VMEM((GPAD,), jnp.float32),
        pltpu.VMEM((GPAD,), jnp.float32),
        pltpu.VMEM((GPAD,), jnp.float32),
        pltpu.VMEM((GPAD,), jnp.float32),
        pltpu.VMEM((GPAD, 4), jnp.float32),
        pltpu.VMEM((64,), jnp.float32),
    ] + [pltpu.SemaphoreType.DMA] * (4 + 2 * CHUNK // 128)
    return functools.partial(
        pl.kernel, mesh=mesh,
        out_type=jax.ShapeDtypeStruct((NW, GPAD, 4), jnp.float32),
        scratch_types=scratch,
        compiler_params=pltpu.CompilerParams(
            needs_layout_passes=False, use_tc_tiling_on_sc=False),
    )(_sc_body)


_UNPAD_ROWS = None


def _unpad_rows():
    global _UNPAD_ROWS
    if _UNPAD_ROWS is None:
        gs = (np.arange(NW + 1, dtype=np.int64) * NUM_GROUPS) // NW
        garr = np.arange(NUM_GROUPS, dtype=np.int64)
        w_of_g = np.searchsorted(gs, garr, side="right") - 1
        _UNPAD_ROWS = jnp.asarray(w_of_g * GPAD + (garr - gs[w_of_g]), dtype=jnp.int32)
    return _UNPAD_ROWS


def kernel(in_arrs, in_slews, c1, c2, rpi, arc_idx_r, arc_idx_f, group,
           unateness, delay_table, slew_table, load_index, slew_index):
    del rpi
    sgn = 1.0 - 2.0 * unateness.astype(jnp.float32)
    axes_tab = jnp.concatenate(
        [slew_index[:, :1] * sgn[:, None], slew_index[:, 1:], load_index], axis=1)
    corner_tab = jnp.concatenate(
        [delay_table[:, :7, :], delay_table[:, 1:, :],
         slew_table[:, :7, :], slew_table[:, 1:, :]], axis=2,
    ).reshape(NUM_ARCS * 7, 32)
    pk = jnp.concatenate(
        [in_arrs, in_slews, c1[:, None], c2[:, None]], axis=1)
    gs = (jnp.arange(NW + 1) * NUM_GROUPS) // NW
    e_bounds = jnp.searchsorted(group, gs.astype(group.dtype), side="left")
    e_bounds = jnp.concatenate(
        [e_bounds.astype(jnp.int32), jnp.full((31,), T_EDGES, jnp.int32)])
    padded = _make_sc_kernel()(
        pk, arc_idx_r.astype(jnp.int32), arc_idx_f.astype(jnp.int32),
        group.astype(jnp.int32), axes_tab, corner_tab, e_bounds)
    return jnp.take(padded.reshape(NW * GPAD, 4), _unpad_rows(), axis=0)
